# Initial kernel scaffold; baseline (speedup 1.0000x reference)
import jax
import jax.numpy as jnp
from jax.experimental import pallas as pl


def kernel(x_user, x_item, ei_u2i, ei_i2u, Wl_u2i, bl_u2i, Wr_u2i, Wl_i2u, bl_i2u, Wr_i2u):
    raise NotImplementedError("write your pallas kernel here")


# speedup vs baseline: 1.1674x; 1.1674x over previous
import functools

import jax
import jax.numpy as jnp
from jax import lax
from jax.experimental import pallas as pl
from jax.experimental.pallas import tpu as pltpu
from jax.experimental.pallas import tpu_sc as plsc

N_NODES = 10000
D = 256
H = 256
DHALF = D // 2
E = 160000
LANES = 128
SUB = 1280
E_PAD = SUB * LANES
N_TILES = 16
SUB_PER_TILE = SUB // N_TILES
HALF_ROWS = 5120
ACC_ROWS = HALF_ROWS + 128
ZERO_PER_TILE = ACC_ROWS // N_TILES
DUMP_PER_TILE = HALF_ROWS // N_TILES
OUT_ROWS = 2 * HALF_ROWS

_MESH = plsc.VectorSubcoreMesh(core_axis_name="c", subcore_axis_name="s")


@functools.partial(
    pl.kernel,
    mesh=_MESH,
    out_type=[
        jax.ShapeDtypeStruct((2, OUT_ROWS, DHALF), jnp.float32),
        jax.ShapeDtypeStruct((OUT_ROWS, DHALF), jnp.float32),
    ],
    scratch_types=[
        pltpu.VMEM((LANES,), jnp.int32),
        pltpu.VMEM((LANES,), jnp.int32),
        pltpu.VMEM((LANES, DHALF), jnp.float32),
        pltpu.VMEM((LANES, DHALF), jnp.float32),
        pltpu.VMEM_SHARED((ACC_ROWS, DHALF), jnp.float32),
        pltpu.SemaphoreType.DMA,
    ],
)
def _sc_segsum(xs_hbm, sidx_hbm, didx_hbm, zrow_hbm, ones_hbm,
               sum_out, cnt_out,
               sidx_v, didx_v, rows_v, ones_v, acc_sh, sem):
    c = lax.axis_index("c")
    s = lax.axis_index("s")
    base0 = s * SUB_PER_TILE

    for p in range(2):
        pltpu.sync_copy(zrow_hbm, acc_sh.at[pl.ds(s * ZERO_PER_TILE,
                                                  ZERO_PER_TILE)])
        plsc.subcore_barrier()

        def body(it, carry):
            e0 = (base0 + it) * LANES
            pltpu.sync_copy(sidx_hbm.at[c, pl.ds(e0, LANES)], sidx_v)
            pltpu.sync_copy(didx_hbm.at[p, pl.ds(e0, LANES)], didx_v)
            pltpu.async_copy(xs_hbm.at[sidx_v], rows_v, sem).wait()
            pltpu.sync_copy(rows_v, acc_sh.at[didx_v], add=True)
            return carry

        lax.fori_loop(0, SUB_PER_TILE, body, 0)
        plsc.subcore_barrier()

        pltpu.sync_copy(
            acc_sh.at[pl.ds(s * DUMP_PER_TILE, DUMP_PER_TILE)],
            sum_out.at[c, pl.ds(p * HALF_ROWS + s * DUMP_PER_TILE,
                                DUMP_PER_TILE)])
        plsc.subcore_barrier()

    pltpu.sync_copy(zrow_hbm, acc_sh.at[pl.ds(s * ZERO_PER_TILE,
                                              ZERO_PER_TILE)])
    pltpu.sync_copy(ones_hbm, ones_v)
    plsc.subcore_barrier()

    def cbody(it, carry):
        e0 = (base0 + it) * LANES
        pltpu.sync_copy(didx_hbm.at[c, pl.ds(e0, LANES)], didx_v)
        pltpu.sync_copy(ones_v, acc_sh.at[didx_v], add=True)
        return carry

    lax.fori_loop(0, SUB_PER_TILE, cbody, 0)
    plsc.subcore_barrier()

    pltpu.sync_copy(
        acc_sh.at[pl.ds(s * DUMP_PER_TILE, DUMP_PER_TILE)],
        cnt_out.at[pl.ds(c * HALF_ROWS + s * DUMP_PER_TILE, DUMP_PER_TILE)])


def _seg_sum(x_src, ei):
    src = ei[0].astype(jnp.int32)
    dst = ei[1].astype(jnp.int32)
    pad = E_PAD - E
    src = jnp.concatenate([src, jnp.zeros((pad,), jnp.int32)])
    dst = jnp.concatenate([dst, jnp.full((pad,), N_NODES, jnp.int32)])
    sidx = jnp.stack([src, src + N_NODES])
    d0 = jnp.where(dst < HALF_ROWS, dst, HALF_ROWS)
    d1 = jnp.where(dst >= HALF_ROWS, dst - HALF_ROWS, HALF_ROWS)
    didx = jnp.stack([d0, d1])
    xs = jnp.concatenate([x_src[:, :DHALF], x_src[:, DHALF:]], axis=0)
    zrow = jnp.zeros((ZERO_PER_TILE, DHALF), jnp.float32)
    ones = jnp.ones((LANES, DHALF), jnp.float32)
    summed2, cnt = _sc_segsum(xs, sidx, didx, zrow, ones)
    summed = jnp.concatenate(
        [summed2[0, :N_NODES], summed2[1, :N_NODES]], axis=1)
    return summed, cnt[:N_NODES, :16]


def _tc_body(sum_ref, cnt_ref, xd_ref, wl_ref, wr_ref, bl_ref, out_ref):
    cnt = cnt_ref[:, 0:1]
    mean = sum_ref[...] * (1.0 / jnp.maximum(cnt, 1.0))
    out_ref[...] = (
        lax.dot_general(mean, wl_ref[...], (((1,), (1,)), ((), ())),
                        preferred_element_type=jnp.float32)
        + lax.dot_general(xd_ref[...], wr_ref[...], (((1,), (1,)), ((), ())),
                          preferred_element_type=jnp.float32)
        + bl_ref[...]
    )


def _linear(summed, cnt16, x_dst, Wl, bl, Wr):
    BLK = 1000
    return pl.pallas_call(
        _tc_body,
        grid=(N_NODES // BLK,),
        in_specs=[
            pl.BlockSpec((BLK, D), lambda i: (i, 0)),
            pl.BlockSpec((BLK, 16), lambda i: (i, 0)),
            pl.BlockSpec((BLK, D), lambda i: (i, 0)),
            pl.BlockSpec((H, D), lambda i: (0, 0)),
            pl.BlockSpec((H, D), lambda i: (0, 0)),
            pl.BlockSpec((1, H), lambda i: (0, 0)),
        ],
        out_specs=pl.BlockSpec((BLK, H), lambda i: (i, 0)),
        out_shape=jax.ShapeDtypeStruct((N_NODES, H), jnp.float32),
    )(summed, cnt16, x_dst, Wl, Wr, bl.reshape(1, H))


def kernel(x_user, x_item, ei_u2i, ei_i2u,
           Wl_u2i, bl_u2i, Wr_u2i, Wl_i2u, bl_i2u, Wr_i2u):
    sum_u2i, cnt_u2i = _seg_sum(x_user, ei_u2i)
    sum_i2u, cnt_i2u = _seg_sum(x_item, ei_i2u)
    out_item = _linear(sum_u2i, cnt_u2i, x_item, Wl_u2i, bl_u2i, Wr_u2i)
    out_user = _linear(sum_i2u, cnt_i2u, x_user, Wl_i2u, bl_i2u, Wr_i2u)
    return (out_user, out_item)


# speedup vs baseline: 1.4465x; 1.2391x over previous
import functools

import jax
import jax.numpy as jnp
from jax import lax
from jax.experimental import pallas as pl
from jax.experimental.pallas import tpu as pltpu
from jax.experimental.pallas import tpu_sc as plsc

N_NODES = 10000
D = 256
H = 256
DHALF = D // 2
E = 160000
LANES = 128
SUB = 1280
E_PAD = SUB * LANES
N_TILES = 16
SUB_PER_TILE = SUB // N_TILES
PAIRS = SUB_PER_TILE // 2
HALF_ROWS = 5120
ACC_ROWS = HALF_ROWS + 128
ZERO_PER_TILE = ACC_ROWS // N_TILES
DUMP_PER_TILE = HALF_ROWS // N_TILES
OUT_ROWS = 2 * HALF_ROWS

_MESH = plsc.VectorSubcoreMesh(core_axis_name="c", subcore_axis_name="s")


@functools.partial(
    pl.kernel,
    mesh=_MESH,
    out_type=[
        jax.ShapeDtypeStruct((2, OUT_ROWS, DHALF), jnp.float32),
        jax.ShapeDtypeStruct((OUT_ROWS, DHALF), jnp.float32),
    ],
    scratch_types=[
        pltpu.VMEM((SUB_PER_TILE, LANES), jnp.int32),
        pltpu.VMEM((SUB_PER_TILE, LANES), jnp.int32),
        pltpu.VMEM((LANES, DHALF), jnp.float32),
        pltpu.VMEM((LANES, DHALF), jnp.float32),
        pltpu.VMEM((LANES, DHALF), jnp.float32),
        pltpu.VMEM_SHARED((ACC_ROWS, DHALF), jnp.float32),
        pltpu.SemaphoreType.DMA,
        pltpu.SemaphoreType.DMA,
    ],
)
def _sc_segsum(xs_hbm, sidx_hbm, didx_hbm, zrow_hbm, ones_hbm,
               sum_out, cnt_out,
               sidx_t, didx_t, rows_a, rows_b, ones_v, acc_sh,
               sem_a, sem_b):
    c = lax.axis_index("c")
    s = lax.axis_index("s")

    def gather(j, rows, sem):
        pltpu.async_copy(xs_hbm.at[sidx_t.at[j]], rows, sem)

    def gwait(rows, sem):
        pltpu.make_async_copy(xs_hbm.at[sidx_t.at[0]], rows, sem).wait()

    def scat(j, rows):
        pltpu.sync_copy(rows, acc_sh.at[didx_t.at[j]], add=True)

    for p in range(2):
        pltpu.sync_copy(zrow_hbm, acc_sh.at[pl.ds(s * ZERO_PER_TILE,
                                                  ZERO_PER_TILE)])
        pltpu.sync_copy(sidx_hbm.at[c, pl.ds(s * SUB_PER_TILE,
                                             SUB_PER_TILE)], sidx_t)
        pltpu.sync_copy(didx_hbm.at[p, pl.ds(s * SUB_PER_TILE,
                                             SUB_PER_TILE)], didx_t)
        plsc.subcore_barrier()

        gather(0, rows_a, sem_a)

        def body(i, carry):
            gather(2 * i + 1, rows_b, sem_b)
            gwait(rows_a, sem_a)
            scat(2 * i, rows_a)

            @pl.when(i < PAIRS - 1)
            def _():
                gather(2 * i + 2, rows_a, sem_a)

            gwait(rows_b, sem_b)
            scat(2 * i + 1, rows_b)
            return carry

        lax.fori_loop(0, PAIRS, body, 0)
        plsc.subcore_barrier()

        pltpu.sync_copy(
            acc_sh.at[pl.ds(s * DUMP_PER_TILE, DUMP_PER_TILE)],
            sum_out.at[c, pl.ds(p * HALF_ROWS + s * DUMP_PER_TILE,
                                DUMP_PER_TILE)])
        plsc.subcore_barrier()

    pltpu.sync_copy(zrow_hbm, acc_sh.at[pl.ds(s * ZERO_PER_TILE,
                                              ZERO_PER_TILE)])
    pltpu.sync_copy(ones_hbm, ones_v)
    pltpu.sync_copy(didx_hbm.at[c, pl.ds(s * SUB_PER_TILE, SUB_PER_TILE)],
                    didx_t)
    plsc.subcore_barrier()

    def cbody(it, carry):
        pltpu.async_copy(ones_v, acc_sh.at[didx_t.at[it]], sem_a, add=True)
        return carry

    lax.fori_loop(0, SUB_PER_TILE, cbody, 0)

    def cdrain(it, carry):
        pltpu.make_async_copy(ones_v, acc_sh.at[didx_t.at[0]], sem_a).wait()
        return carry

    lax.fori_loop(0, SUB_PER_TILE, cdrain, 0)
    plsc.subcore_barrier()

    pltpu.sync_copy(
        acc_sh.at[pl.ds(s * DUMP_PER_TILE, DUMP_PER_TILE)],
        cnt_out.at[pl.ds(c * HALF_ROWS + s * DUMP_PER_TILE, DUMP_PER_TILE)])


def _seg_sum(x_src, ei):
    src = ei[0].astype(jnp.int32)
    dst = ei[1].astype(jnp.int32)
    pad = E_PAD - E
    src = jnp.concatenate([src, jnp.zeros((pad,), jnp.int32)])
    dst = jnp.concatenate([dst, jnp.full((pad,), N_NODES, jnp.int32)])
    sidx = jnp.stack([src, src + N_NODES]).reshape(2, SUB, LANES)
    d0 = jnp.where(dst < HALF_ROWS, dst, HALF_ROWS)
    d1 = jnp.where(dst >= HALF_ROWS, dst - HALF_ROWS, HALF_ROWS)
    didx = jnp.stack([d0, d1]).reshape(2, SUB, LANES)
    xs = jnp.concatenate([x_src[:, :DHALF], x_src[:, DHALF:]], axis=0)
    zrow = jnp.zeros((ZERO_PER_TILE, DHALF), jnp.float32)
    ones = jnp.ones((LANES, DHALF), jnp.float32)
    summed2, cnt = _sc_segsum(xs, sidx, didx, zrow, ones)
    summed = jnp.concatenate(
        [summed2[0, :N_NODES], summed2[1, :N_NODES]], axis=1)
    return summed, cnt[:N_NODES, :16]


def _tc_body(sum_ref, cnt_ref, xd_ref, wl_ref, wr_ref, bl_ref, out_ref):
    cnt = cnt_ref[:, 0:1]
    mean = sum_ref[...] * (1.0 / jnp.maximum(cnt, 1.0))
    out_ref[...] = (
        lax.dot_general(mean, wl_ref[...], (((1,), (1,)), ((), ())),
                        preferred_element_type=jnp.float32)
        + lax.dot_general(xd_ref[...], wr_ref[...], (((1,), (1,)), ((), ())),
                          preferred_element_type=jnp.float32)
        + bl_ref[...]
    )


def _linear(summed, cnt16, x_dst, Wl, bl, Wr):
    BLK = 1000
    return pl.pallas_call(
        _tc_body,
        grid=(N_NODES // BLK,),
        in_specs=[
            pl.BlockSpec((BLK, D), lambda i: (i, 0)),
            pl.BlockSpec((BLK, 16), lambda i: (i, 0)),
            pl.BlockSpec((BLK, D), lambda i: (i, 0)),
            pl.BlockSpec((H, D), lambda i: (0, 0)),
            pl.BlockSpec((H, D), lambda i: (0, 0)),
            pl.BlockSpec((1, H), lambda i: (0, 0)),
        ],
        out_specs=pl.BlockSpec((BLK, H), lambda i: (i, 0)),
        out_shape=jax.ShapeDtypeStruct((N_NODES, H), jnp.float32),
    )(summed, cnt16, x_dst, Wl, Wr, bl.reshape(1, H))


def kernel(x_user, x_item, ei_u2i, ei_i2u,
           Wl_u2i, bl_u2i, Wr_u2i, Wl_i2u, bl_i2u, Wr_i2u):
    sum_u2i, cnt_u2i = _seg_sum(x_user, ei_u2i)
    sum_i2u, cnt_i2u = _seg_sum(x_item, ei_i2u)
    out_item = _linear(sum_u2i, cnt_u2i, x_item, Wl_u2i, bl_u2i, Wr_u2i)
    out_user = _linear(sum_i2u, cnt_i2u, x_user, Wl_i2u, bl_i2u, Wr_i2u)
    return (out_user, out_item)


# speedup vs baseline: 1.4885x; 1.0290x over previous
import functools

import jax
import jax.numpy as jnp
from jax import lax
from jax.experimental import pallas as pl
from jax.experimental.pallas import tpu as pltpu
from jax.experimental.pallas import tpu_sc as plsc

N_NODES = 10000
D = 256
H = 256
DHALF = D // 2
E = 160000
LANES = 128
SUB = 1280
E_PAD = SUB * LANES
N_TILES = 16
SUB_PER_TILE = SUB // N_TILES
NBUF = 4
NBATCH = SUB_PER_TILE // NBUF
HALF_ROWS = 5120
ACC_ROWS = HALF_ROWS + 128
ZERO_PER_TILE = ACC_ROWS // N_TILES
DUMP_PER_TILE = HALF_ROWS // N_TILES
OUT_ROWS = 2 * HALF_ROWS

_MESH = plsc.VectorSubcoreMesh(core_axis_name="c", subcore_axis_name="s")


@functools.partial(
    pl.kernel,
    mesh=_MESH,
    out_type=[
        jax.ShapeDtypeStruct((2, OUT_ROWS, DHALF), jnp.float32),
        jax.ShapeDtypeStruct((OUT_ROWS, DHALF), jnp.float32),
    ],
    scratch_types=[
        pltpu.VMEM((SUB_PER_TILE, LANES), jnp.int32),
        pltpu.VMEM((SUB_PER_TILE, LANES), jnp.int32),
    ] + [pltpu.VMEM((LANES, DHALF), jnp.float32) for _ in range(NBUF)]
      + [pltpu.VMEM_SHARED((ACC_ROWS, DHALF), jnp.float32)]
      + [pltpu.SemaphoreType.DMA for _ in range(2 * NBUF)],
)
def _sc_segsum(xs_hbm, sidx_hbm, didx_hbm, zrow_hbm, ones_hbm,
               sum_out, cnt_out,
               sidx_t, didx_t, *rest):
    rows = list(rest[:NBUF])
    acc_sh = rest[NBUF]
    gsem = list(rest[NBUF + 1:NBUF + 1 + NBUF])
    ssem = list(rest[NBUF + 1 + NBUF:])
    c = lax.axis_index("c")
    s = lax.axis_index("s")

    def gather(j, b):
        pltpu.async_copy(xs_hbm.at[sidx_t.at[j]], rows[b], gsem[b])

    def gwait(b):
        pltpu.make_async_copy(xs_hbm.at[sidx_t.at[0]], rows[b],
                              gsem[b]).wait()

    def scat(j, b):
        pltpu.async_copy(rows[b], acc_sh.at[didx_t.at[j]], ssem[b],
                         add=True)

    def swait(b):
        pltpu.make_async_copy(rows[b], acc_sh.at[didx_t.at[0]],
                              ssem[b]).wait()

    for p in range(2):
        pltpu.sync_copy(zrow_hbm, acc_sh.at[pl.ds(s * ZERO_PER_TILE,
                                                  ZERO_PER_TILE)])
        pltpu.sync_copy(sidx_hbm.at[c, pl.ds(s * SUB_PER_TILE,
                                             SUB_PER_TILE)], sidx_t)
        pltpu.sync_copy(didx_hbm.at[p, pl.ds(s * SUB_PER_TILE,
                                             SUB_PER_TILE)], didx_t)
        plsc.subcore_barrier()

        for b in range(NBUF):
            gather(b, b)

        def body(i, carry):
            base = i * NBUF
            for b in range(NBUF):
                gwait(b)
                scat(base + b, b)
            for b in range(NBUF):
                @pl.when(i < NBATCH - 1)
                def _():
                    swait(b)
                    gather(base + NBUF + b, b)
            return carry

        lax.fori_loop(0, NBATCH, body, 0)
        for b in range(NBUF):
            swait(b)
        plsc.subcore_barrier()

        pltpu.sync_copy(
            acc_sh.at[pl.ds(s * DUMP_PER_TILE, DUMP_PER_TILE)],
            sum_out.at[c, pl.ds(p * HALF_ROWS + s * DUMP_PER_TILE,
                                DUMP_PER_TILE)])
        plsc.subcore_barrier()

    pltpu.sync_copy(zrow_hbm, acc_sh.at[pl.ds(s * ZERO_PER_TILE,
                                              ZERO_PER_TILE)])
    pltpu.sync_copy(ones_hbm, rows[0])
    pltpu.sync_copy(didx_hbm.at[c, pl.ds(s * SUB_PER_TILE, SUB_PER_TILE)],
                    didx_t)
    plsc.subcore_barrier()

    def cbody(it, carry):
        pltpu.async_copy(rows[0], acc_sh.at[didx_t.at[it]], ssem[0],
                         add=True)
        return carry

    lax.fori_loop(0, SUB_PER_TILE, cbody, 0)

    def cdrain(it, carry):
        pltpu.make_async_copy(rows[0], acc_sh.at[didx_t.at[0]],
                              ssem[0]).wait()
        return carry

    lax.fori_loop(0, SUB_PER_TILE, cdrain, 0)
    plsc.subcore_barrier()

    pltpu.sync_copy(
        acc_sh.at[pl.ds(s * DUMP_PER_TILE, DUMP_PER_TILE)],
        cnt_out.at[pl.ds(c * HALF_ROWS + s * DUMP_PER_TILE, DUMP_PER_TILE)])


def _seg_sum(x_src, ei):
    src = ei[0].astype(jnp.int32)
    dst = ei[1].astype(jnp.int32)
    pad = E_PAD - E
    src = jnp.concatenate([src, jnp.zeros((pad,), jnp.int32)])
    dst = jnp.concatenate([dst, jnp.full((pad,), N_NODES, jnp.int32)])
    sidx = jnp.stack([src, src + N_NODES]).reshape(2, SUB, LANES)
    d0 = jnp.where(dst < HALF_ROWS, dst, HALF_ROWS)
    d1 = jnp.where(dst >= HALF_ROWS, dst - HALF_ROWS, HALF_ROWS)
    didx = jnp.stack([d0, d1]).reshape(2, SUB, LANES)
    xs = jnp.concatenate([x_src[:, :DHALF], x_src[:, DHALF:]], axis=0)
    zrow = jnp.zeros((ZERO_PER_TILE, DHALF), jnp.float32)
    ones = jnp.ones((LANES, DHALF), jnp.float32)
    summed2, cnt = _sc_segsum(xs, sidx, didx, zrow, ones)
    summed = jnp.concatenate(
        [summed2[0, :N_NODES], summed2[1, :N_NODES]], axis=1)
    return summed, cnt[:N_NODES, :16]


def _tc_body(sum_ref, cnt_ref, xd_ref, wl_ref, wr_ref, bl_ref, out_ref):
    cnt = cnt_ref[:, 0:1]
    mean = sum_ref[...] * (1.0 / jnp.maximum(cnt, 1.0))
    out_ref[...] = (
        lax.dot_general(mean, wl_ref[...], (((1,), (1,)), ((), ())),
                        preferred_element_type=jnp.float32)
        + lax.dot_general(xd_ref[...], wr_ref[...], (((1,), (1,)), ((), ())),
                          preferred_element_type=jnp.float32)
        + bl_ref[...]
    )


def _linear(summed, cnt16, x_dst, Wl, bl, Wr):
    BLK = 1000
    return pl.pallas_call(
        _tc_body,
        grid=(N_NODES // BLK,),
        in_specs=[
            pl.BlockSpec((BLK, D), lambda i: (i, 0)),
            pl.BlockSpec((BLK, 16), lambda i: (i, 0)),
            pl.BlockSpec((BLK, D), lambda i: (i, 0)),
            pl.BlockSpec((H, D), lambda i: (0, 0)),
            pl.BlockSpec((H, D), lambda i: (0, 0)),
            pl.BlockSpec((1, H), lambda i: (0, 0)),
        ],
        out_specs=pl.BlockSpec((BLK, H), lambda i: (i, 0)),
        out_shape=jax.ShapeDtypeStruct((N_NODES, H), jnp.float32),
    )(summed, cnt16, x_dst, Wl, Wr, bl.reshape(1, H))


def kernel(x_user, x_item, ei_u2i, ei_i2u,
           Wl_u2i, bl_u2i, Wr_u2i, Wl_i2u, bl_i2u, Wr_i2u):
    sum_u2i, cnt_u2i = _seg_sum(x_user, ei_u2i)
    sum_i2u, cnt_i2u = _seg_sum(x_item, ei_i2u)
    out_item = _linear(sum_u2i, cnt_u2i, x_item, Wl_u2i, bl_u2i, Wr_u2i)
    out_user = _linear(sum_i2u, cnt_i2u, x_user, Wl_i2u, bl_i2u, Wr_i2u)
    return (out_user, out_item)


# speedup vs baseline: 2.3877x; 1.6041x over previous
import functools

import jax
import jax.numpy as jnp
from jax import lax
from jax.experimental import pallas as pl
from jax.experimental.pallas import tpu as pltpu
from jax.experimental.pallas import tpu_sc as plsc

N_NODES = 10000
D = 256
H = 256
DHALF = D // 2
E = 160000
LANES = 128
SUB = 1280
E_PAD = SUB * LANES
N_TILES = 16
SUB_PER_TILE = SUB // N_TILES
IDX_CHUNK = 40
NCHUNK = SUB_PER_TILE // IDX_CHUNK
NBUF = 2
NBATCH = IDX_CHUNK // NBUF
ACC_ROWS = 10240
ZERO_PER_TILE = ACC_ROWS // N_TILES
DUMP_PER_TILE = ACC_ROWS // N_TILES
HALF_ROWS = ACC_ROWS // 2
CNT_TRASH = HALF_ROWS

_MESH = plsc.VectorSubcoreMesh(core_axis_name="c", subcore_axis_name="s")


@functools.partial(
    pl.kernel,
    mesh=_MESH,
    out_type=[
        jax.ShapeDtypeStruct((2, ACC_ROWS, DHALF), jnp.float32),
        jax.ShapeDtypeStruct((ACC_ROWS, DHALF), jnp.float32),
    ],
    scratch_types=[
        pltpu.VMEM((IDX_CHUNK, LANES), jnp.int32),
        pltpu.VMEM((IDX_CHUNK, LANES), jnp.int32),
    ] + [pltpu.VMEM((LANES, DHALF), jnp.float32) for _ in range(NBUF)]
      + [pltpu.VMEM_SHARED((ACC_ROWS, DHALF), jnp.float32)]
      + [pltpu.SemaphoreType.DMA for _ in range(2 * NBUF)],
)
def _sc_segsum(xs_hbm, sidx_hbm, didx_hbm, cidx_hbm, zrow_hbm, ones_hbm,
               sum_out, cnt_out,
               sidx_t, didx_t, *rest):
    rows = list(rest[:NBUF])
    acc_sh = rest[NBUF]
    gsem = list(rest[NBUF + 1:NBUF + 1 + NBUF])
    ssem = list(rest[NBUF + 1 + NBUF:])
    c = lax.axis_index("c")
    s = lax.axis_index("s")

    def gather(j, b):
        pltpu.async_copy(xs_hbm.at[sidx_t.at[j]], rows[b], gsem[b])

    def gwait(b):
        pltpu.make_async_copy(xs_hbm.at[sidx_t.at[0]], rows[b],
                              gsem[b]).wait()

    def scat(j, b):
        pltpu.async_copy(rows[b], acc_sh.at[didx_t.at[j]], ssem[b],
                         add=True)

    def swait(b):
        pltpu.make_async_copy(rows[b], acc_sh.at[didx_t.at[0]],
                              ssem[b]).wait()

    pltpu.sync_copy(zrow_hbm, acc_sh.at[pl.ds(s * ZERO_PER_TILE,
                                              ZERO_PER_TILE)])
    plsc.subcore_barrier()

    for chunk in range(NCHUNK):
        base_sub = s * SUB_PER_TILE + chunk * IDX_CHUNK
        pltpu.sync_copy(sidx_hbm.at[c, pl.ds(base_sub, IDX_CHUNK)], sidx_t)
        pltpu.sync_copy(didx_hbm.at[pl.ds(base_sub, IDX_CHUNK)], didx_t)

        for b in range(NBUF):
            gather(b, b)

        def body(i, carry):
            base = i * NBUF
            for b in range(NBUF):
                gwait(b)
                scat(base + b, b)
            for b in range(NBUF):
                @pl.when(i < NBATCH - 1)
                def _():
                    swait(b)
                    gather(base + NBUF + b, b)
            return carry

        lax.fori_loop(0, NBATCH, body, 0)
        for b in range(NBUF):
            swait(b)

    plsc.subcore_barrier()
    pltpu.sync_copy(
        acc_sh.at[pl.ds(s * DUMP_PER_TILE, DUMP_PER_TILE)],
        sum_out.at[c, pl.ds(s * DUMP_PER_TILE, DUMP_PER_TILE)])
    plsc.subcore_barrier()

    pltpu.sync_copy(zrow_hbm, acc_sh.at[pl.ds(s * ZERO_PER_TILE,
                                              ZERO_PER_TILE)])
    pltpu.sync_copy(ones_hbm, rows[0])
    plsc.subcore_barrier()

    for chunk in range(NCHUNK):
        base_sub = s * SUB_PER_TILE + chunk * IDX_CHUNK
        pltpu.sync_copy(cidx_hbm.at[c, pl.ds(base_sub, IDX_CHUNK)], didx_t)

        def cbody(it, carry):
            pltpu.async_copy(rows[0], acc_sh.at[didx_t.at[it]], ssem[0],
                             add=True)
            return carry

        lax.fori_loop(0, IDX_CHUNK, cbody, 0)

        def cdrain(it, carry):
            pltpu.make_async_copy(rows[0], acc_sh.at[didx_t.at[0]],
                                  ssem[0]).wait()
            return carry

        lax.fori_loop(0, IDX_CHUNK, cdrain, 0)

    plsc.subcore_barrier()

    @pl.when(s < N_TILES // 2)
    def _():
        pltpu.sync_copy(
            acc_sh.at[pl.ds(s * DUMP_PER_TILE, DUMP_PER_TILE)],
            cnt_out.at[pl.ds(c * HALF_ROWS + s * DUMP_PER_TILE,
                             DUMP_PER_TILE)])


def _seg_sum(x_src, ei):
    src = ei[0].astype(jnp.int32)
    dst = ei[1].astype(jnp.int32)
    pad = E_PAD - E
    src = jnp.concatenate([src, jnp.zeros((pad,), jnp.int32)])
    dst = jnp.concatenate([dst, jnp.full((pad,), N_NODES, jnp.int32)])
    sidx = jnp.stack([src, src + N_NODES]).reshape(2, SUB, LANES)
    didx = dst.reshape(SUB, LANES)
    d0 = jnp.where(dst < HALF_ROWS, dst, CNT_TRASH)
    d1 = jnp.where(dst >= HALF_ROWS, dst - HALF_ROWS, CNT_TRASH)
    cidx = jnp.stack([d0, d1]).reshape(2, SUB, LANES)
    xs = jnp.concatenate([x_src[:, :DHALF], x_src[:, DHALF:]], axis=0)
    zrow = jnp.zeros((ZERO_PER_TILE, DHALF), jnp.float32)
    ones = jnp.ones((LANES, DHALF), jnp.float32)
    summed2, cnt = _sc_segsum(xs, sidx, didx, cidx, zrow, ones)
    summed = jnp.concatenate(
        [summed2[0, :N_NODES], summed2[1, :N_NODES]], axis=1)
    return summed, cnt[:N_NODES, :16]


def _tc_body(sum_ref, cnt_ref, xd_ref, wl_ref, wr_ref, bl_ref, out_ref):
    cnt = cnt_ref[:, 0:1]
    mean = sum_ref[...] * (1.0 / jnp.maximum(cnt, 1.0))
    out_ref[...] = (
        lax.dot_general(mean, wl_ref[...], (((1,), (1,)), ((), ())),
                        preferred_element_type=jnp.float32)
        + lax.dot_general(xd_ref[...], wr_ref[...], (((1,), (1,)), ((), ())),
                          preferred_element_type=jnp.float32)
        + bl_ref[...]
    )


def _linear(summed, cnt16, x_dst, Wl, bl, Wr):
    BLK = 1000
    return pl.pallas_call(
        _tc_body,
        grid=(N_NODES // BLK,),
        in_specs=[
            pl.BlockSpec((BLK, D), lambda i: (i, 0)),
            pl.BlockSpec((BLK, 16), lambda i: (i, 0)),
            pl.BlockSpec((BLK, D), lambda i: (i, 0)),
            pl.BlockSpec((H, D), lambda i: (0, 0)),
            pl.BlockSpec((H, D), lambda i: (0, 0)),
            pl.BlockSpec((1, H), lambda i: (0, 0)),
        ],
        out_specs=pl.BlockSpec((BLK, H), lambda i: (i, 0)),
        out_shape=jax.ShapeDtypeStruct((N_NODES, H), jnp.float32),
    )(summed, cnt16, x_dst, Wl, Wr, bl.reshape(1, H))


def kernel(x_user, x_item, ei_u2i, ei_i2u,
           Wl_u2i, bl_u2i, Wr_u2i, Wl_i2u, bl_i2u, Wr_i2u):
    sum_u2i, cnt_u2i = _seg_sum(x_user, ei_u2i)
    sum_i2u, cnt_i2u = _seg_sum(x_item, ei_i2u)
    out_item = _linear(sum_u2i, cnt_u2i, x_item, Wl_u2i, bl_u2i, Wr_u2i)
    out_user = _linear(sum_i2u, cnt_i2u, x_user, Wl_i2u, bl_i2u, Wr_i2u)
    return (out_user, out_item)
